# Initial kernel scaffold; baseline (speedup 1.0000x reference)
import jax
import jax.numpy as jnp
from jax.experimental import pallas as pl


def kernel(word_mask, word_repr, candidates_idx, candidate_label, anchor_loc, anchor_label, anchor_cls, batch_candidate_num, key_candidates, key_candidate_mask, key_candidate_len, key_candidate_loc, W, b):
    raise NotImplementedError("write your pallas kernel here")


# speedup vs baseline: 4.0226x; 4.0226x over previous
import functools

import jax
import jax.numpy as jnp
from jax import lax
from jax.experimental import pallas as pl
from jax.experimental.pallas import tpu as pltpu
from jax.experimental.pallas import tpu_sc as plsc

_B, _S, _A, _D, _C, _K = 16, 512, 8, 512, 1024, 34
_SA = _S * _A
_NW = 32
_ROWS_W = (_B * _C) // _NW
_CHUNK = 64
_NCHUNK = _ROWS_W // _CHUNK
_ROWS_TC = 512
_KP = 128


def _gather_body(wr_hbm, idx_hbm, out_hbm, idx_v, buf0, buf1, sem0, sem1):
    wid = lax.axis_index("s") * 2 + lax.axis_index("c")
    base_row = wid * _ROWS_W
    row_off = (wid // 2) * _SA
    pltpu.sync_copy(idx_hbm.at[pl.ds(base_row, _ROWS_W)], idx_v)
    for j in range(_ROWS_W // 16):
        sl = pl.ds(j * 16, 16)
        idx_v[sl] = idx_v[sl] + row_off
    bufs = (buf0, buf1)
    sems = (sem0, sem1)
    copies = [
        pltpu.make_async_copy(
            wr_hbm.at[idx_v.at[pl.ds(g * _CHUNK, _CHUNK)]],
            bufs[g % 2], sems[g % 2])
        for g in range(_NCHUNK)
    ]
    copies[0].start()
    for g in range(_NCHUNK):
        if g + 1 < _NCHUNK:
            copies[g + 1].start()
        copies[g].wait()
        pltpu.sync_copy(bufs[g % 2],
                        out_hbm.at[pl.ds(base_row + g * _CHUNK, _CHUNK)])


def _sc_gather(wr_flat, cidx_flat):
    mesh = plsc.VectorSubcoreMesh(core_axis_name="c", subcore_axis_name="s")
    kern = functools.partial(
        pl.kernel, _gather_body, mesh=mesh,
        out_type=jax.ShapeDtypeStruct((_B * _C, _D), jnp.float32),
        scratch_types=[
            pltpu.VMEM((_ROWS_W,), jnp.int32),
            pltpu.VMEM((_CHUNK, _D), jnp.float32),
            pltpu.VMEM((_CHUNK, _D), jnp.float32),
            pltpu.SemaphoreType.DMA,
            pltpu.SemaphoreType.DMA,
        ],
    )()
    return kern(wr_flat, cidx_flat)


def _classify_body(x_ref, wt_ref, bias_ref, lab_ref, cidx_ref,
                   loss_ref, bcp_ref, mask_ref):
    i = pl.program_id(0)
    x = x_ref[...]
    logits = jnp.dot(x, wt_ref[...], preferred_element_type=jnp.float32)
    logits = logits + bias_ref[...]
    kio = lax.broadcasted_iota(jnp.int32, (_ROWS_TC, _KP), 1)
    valid = kio < _K
    lg = jnp.where(valid, logits, -1e30)
    m = jnp.max(lg, axis=-1, keepdims=True)
    e = jnp.where(valid, jnp.exp(lg - m), 0.0)
    logz = m + jnp.log(jnp.sum(e, axis=-1, keepdims=True))
    lab = lab_ref[...]
    ll = jnp.sum(jnp.where(kio == lab, lg, 0.0), axis=-1, keepdims=True)
    bcp = jnp.min(jnp.where(lg >= m, kio, _KP), axis=-1, keepdims=True)
    bcp_ref[...] = bcp.astype(jnp.int32)
    mask_ref[...] = (cidx_ref[...] != 0).astype(jnp.float32)

    @pl.when(i == 0)
    def _():
        loss_ref[...] = jnp.zeros_like(loss_ref)

    loss_ref[...] += jnp.sum(logz - ll, keepdims=True)

    @pl.when(i == pl.num_programs(0) - 1)
    def _():
        loss_ref[...] = loss_ref[...] / float(_B * _C)


def _tc_classify(cand_flat, wt, bias, lab2, cidx2):
    grid = (_B * _C) // _ROWS_TC
    return pl.pallas_call(
        _classify_body,
        grid=(grid,),
        in_specs=[
            pl.BlockSpec((_ROWS_TC, _D), lambda i: (i, 0)),
            pl.BlockSpec((_D, _KP), lambda i: (0, 0)),
            pl.BlockSpec((1, _KP), lambda i: (0, 0)),
            pl.BlockSpec((_ROWS_TC, 1), lambda i: (i, 0)),
            pl.BlockSpec((_ROWS_TC, 1), lambda i: (i, 0)),
        ],
        out_specs=[
            pl.BlockSpec((1, 1), lambda i: (0, 0)),
            pl.BlockSpec((_ROWS_TC, 1), lambda i: (i, 0)),
            pl.BlockSpec((_ROWS_TC, 1), lambda i: (i, 0)),
        ],
        out_shape=[
            jax.ShapeDtypeStruct((1, 1), jnp.float32),
            jax.ShapeDtypeStruct((_B * _C, 1), jnp.int32),
            jax.ShapeDtypeStruct((_B * _C, 1), jnp.float32),
        ],
    )(cand_flat, wt, bias, lab2, cidx2)


def _predict_body(idx_hbm, bcp_hbm, out_hbm, idx_v, val_v, table):
    wid = lax.axis_index("s") * 2 + lax.axis_index("c")

    @pl.when(wid < _B)
    def _():
        pltpu.sync_copy(idx_hbm.at[wid], idx_v.at[pl.ds(0, _C)])
        pltpu.sync_copy(bcp_hbm.at[wid], val_v)
        zv = jnp.zeros((16,), jnp.int32)

        def zbody(t, carry):
            table[pl.ds(t * 16, 16)] = zv
            return carry

        lax.fori_loop(0, _SA // 16, zbody, 0)
        iota = lax.iota(jnp.int32, 16)

        def gbody(g, carry):
            base = g * 16
            iv = idx_v[pl.ds(base, 16)]
            vv = val_v[pl.ds(base, 16)]
            keep = jnp.ones((16,), jnp.bool_)
            for s in range(1, 16):
                shifted = idx_v[pl.ds(base + s, 16)]
                dup = (iv == shifted) & (iota < (16 - s))
                keep = keep & jnp.logical_not(dup)
            plsc.store_scatter(table, [iv], vv, mask=keep)
            return carry

        lax.fori_loop(0, _C // 16, gbody, 0)
        pltpu.sync_copy(table, out_hbm.at[wid])


def _sc_predict(cidx, bcp):
    mesh = plsc.VectorSubcoreMesh(core_axis_name="c", subcore_axis_name="s")
    kern = functools.partial(
        pl.kernel, _predict_body, mesh=mesh,
        out_type=jax.ShapeDtypeStruct((_B, _SA), jnp.int32),
        scratch_types=[
            pltpu.VMEM((_C + 16,), jnp.int32),
            pltpu.VMEM((_C,), jnp.int32),
            pltpu.VMEM((_SA,), jnp.int32),
        ],
        compiler_params=pltpu.CompilerParams(needs_layout_passes=False),
    )()
    return kern(cidx, bcp)


def kernel(word_mask, word_repr, candidates_idx, candidate_label, anchor_loc,
           anchor_label, anchor_cls, batch_candidate_num, key_candidates,
           key_candidate_mask, key_candidate_len, key_candidate_loc, W, b):
    wr_flat = word_repr.reshape(_B * _SA, _D)
    cidx_flat = candidates_idx.reshape(-1)
    cand_flat = _sc_gather(wr_flat, cidx_flat)

    wt = jnp.zeros((_D, _KP), jnp.float32).at[:, :_K].set(W.T)
    bias = jnp.zeros((1, _KP), jnp.float32).at[0, :_K].set(b)
    lab2 = candidate_label.reshape(_B * _C, 1)
    cidx2 = cidx_flat.reshape(_B * _C, 1)
    loss, bcp2, mask2 = _tc_classify(cand_flat, wt, bias, lab2, cidx2)

    bcp = bcp2.reshape(_B, _C)
    pred = _sc_predict(candidates_idx, bcp)

    return (loss[0, 0], pred.reshape(_B, _S, _A),
            cand_flat.reshape(_B, _C, _D), candidate_label,
            bcp, mask2.reshape(_B, _C))
